# baseline (speedup 1.0000x reference)
import jax
import jax.numpy as jnp
from jax import lax
from jax.experimental import pallas as pl
from jax.experimental.pallas import tpu as pltpu
from jax.experimental.pallas import tpu_sc as plsc

VOCAB = 400001
DIM = 100
BATCH = 4096
HIST = 200

DPAD = 128
VB = 16384
NBLK = (VOCAB + VB - 1) // VB
VPAD = NBLK * VB

NC, NS = 2, 16
NW = NC * NS

HALF = HIST // 2
N_SLICE = 8
NBAT = BATCH // N_SLICE
BATCH_PER_W = NBAT // NW
SUPER = BATCH_PER_W // 2


def _transpose_pad(tableT):
    def body(in_ref, out_ref):
        out_ref[:, pl.ds(0, DIM)] = in_ref[...].T

    return pl.pallas_call(
        body,
        grid=(NBLK,),
        in_specs=[pl.BlockSpec((DIM, VB), lambda i: (0, i))],
        out_specs=pl.BlockSpec((VB, DPAD), lambda i: (i, 0)),
        out_shape=jax.ShapeDtypeStruct((VPAD, DPAD), jnp.float32),
    )(tableT)


BB, HB = 512, 8
GB, GH = BATCH // BB, HIST // HB


def _relayout(o):
    def body(in_ref, out_ref):
        for hh in range(HB):
            out_ref[:, hh, :] = in_ref[:, hh, :].T

    oT = pl.pallas_call(
        body,
        grid=(GB, GH),
        in_specs=[pl.BlockSpec((BB, HB, DIM), lambda i, j: (i, j, 0))],
        out_specs=pl.BlockSpec((DIM, HB, BB), lambda i, j: (0, j, i)),
        out_shape=jax.ShapeDtypeStruct((DIM, HIST, BATCH), jnp.float32),
    )(o)
    return jnp.transpose(oT, (2, 1, 0))


def _relayout_slice(acc, o_s, s):
    def body(acc_ref, in_ref, out_ref):
        for hh in range(HB):
            out_ref[:, hh, :] = in_ref[:, hh, :].T

    col0 = s * (NBAT // BB)
    return pl.pallas_call(
        body,
        grid=(NBAT // BB, GH),
        in_specs=[pl.BlockSpec(memory_space=pl.ANY),
                  pl.BlockSpec((BB, HB, DIM), lambda i, j: (i, j, 0))],
        out_specs=pl.BlockSpec((DIM, HB, BB), lambda i, j: (0, j, col0 + i)),
        out_shape=jax.ShapeDtypeStruct((DIM, HIST, BATCH), jnp.float32),
        input_output_aliases={0: 0},
    )(acc, o_s)


def _relayout_first(o_s):
    def body(in_ref, out_ref):
        for hh in range(HB):
            out_ref[:, hh, :] = in_ref[:, hh, :].T

    return pl.pallas_call(
        body,
        grid=(NBAT // BB, GH),
        in_specs=[pl.BlockSpec((BB, HB, DIM), lambda i, j: (i, j, 0))],
        out_specs=pl.BlockSpec((DIM, HB, BB), lambda i, j: (0, j, i)),
        out_shape=jax.ShapeDtypeStruct((DIM, HIST, BATCH), jnp.float32),
    )(o_s)


def _make_gather():
    mesh = plsc.VectorSubcoreMesh(core_axis_name="c", subcore_axis_name="s")

    @pl.kernel(
        out_type=jax.ShapeDtypeStruct((NBAT, HIST, DIM), jnp.float32),
        mesh=mesh,
        scratch_types=[
            pltpu.VMEM((2 * BATCH_PER_W, DPAD), jnp.int32),
            pltpu.VMEM((4, HALF, DPAD), jnp.float32),
            pltpu.VMEM((HIST, DIM), jnp.float32),
            pltpu.SemaphoreType.DMA((4,)),
            pltpu.SemaphoreType.DMA,
        ],
    )
    def emb_kernel(x_hbm, table_hbm, out_hbm, idx_v, gbuf, sbuf, gsem, wsem):
        wid = lax.axis_index("s") * NC + lax.axis_index("c")
        bat0 = wid * BATCH_PER_W
        pltpu.sync_copy(x_hbm.at[pl.ds(2 * bat0, 2 * BATCH_PER_W)], idx_v)
        tail_mask = lax.iota(jnp.int32, 16) >= 12

        def gather(k, half, s):
            pltpu.async_copy(
                table_hbm.at[idx_v.at[2 * k + half, pl.ds(0, HALF)]],
                gbuf.at[s], gsem.at[s])

        def gather_wait(s):
            pltpu.make_async_copy(
                table_hbm.at[idx_v.at[0, pl.ds(0, HALF)]], gbuf.at[s], gsem.at[s]
            ).wait()

        def compact(s, half):
            def rows4(i4, carry):
                for r in range(4):
                    i = i4 * 4 + r
                    o = half * HALF + i
                    for k in range(6):
                        sbuf[o, pl.ds(16 * k, 16)] = gbuf[s, i, pl.ds(16 * k, 16)]
                    tail = gbuf[s, i, pl.ds(84, 16)]
                    cur = sbuf[o, pl.ds(84, 16)]
                    sbuf[o, pl.ds(84, 16)] = jnp.where(tail_mask, tail, cur)
                return carry

            lax.fori_loop(0, HALF // 4, rows4, 0)

        def write(k):
            pltpu.async_copy(sbuf, out_hbm.at[bat0 + k], wsem)

        def write_wait():
            pltpu.make_async_copy(sbuf, out_hbm.at[0], wsem).wait()

        def batch_body(k, b, first, last):
            s0, s1 = 2 * b, 2 * b + 1
            gather_wait(s0)
            if not first:
                write_wait()
            compact(s0, 0)
            if not last:
                gather(k + 2, 0, s0)
            gather_wait(s1)
            compact(s1, 1)
            if not last:
                gather(k + 2, 1, s1)
            write(k)

        for b in range(2):
            for half in range(2):
                gather(b, half, 2 * b + half)
        batch_body(0, 0, True, False)
        batch_body(1, 1, False, False)

        def round_(g, carry):
            for b in range(2):
                batch_body(g * 2 + b, b, False, False)
            return carry

        lax.fori_loop(1, SUPER - 1, round_, 0)

        batch_body(BATCH_PER_W - 2, 0, False, True)
        batch_body(BATCH_PER_W - 1, 1, False, True)
        write_wait()

    return emb_kernel


_gather = _make_gather()


@jax.jit
def kernel(x, table):
    tablep = _transpose_pad(jnp.transpose(table))
    x2 = jnp.pad(x.reshape(2 * BATCH, HALF).astype(jnp.int32),
                 ((0, 0), (0, DPAD - HALF)))
    if N_SLICE == 1:
        return _relayout(_gather(x2, tablep))
    acc = None
    for s in range(N_SLICE):
        o_s = _gather(lax.slice_in_dim(x2, 2 * s * NBAT, 2 * (s + 1) * NBAT),
                      tablep)
        acc = _relayout_first(o_s) if s == 0 else _relayout_slice(acc, o_s, s)
    return jnp.transpose(acc, (2, 1, 0))


# speedup vs baseline: 1.0342x; 1.0342x over previous
import jax
import jax.numpy as jnp
from jax import lax
from jax.experimental import pallas as pl
from jax.experimental.pallas import tpu as pltpu
from jax.experimental.pallas import tpu_sc as plsc

VOCAB = 400001
DIM = 100
BATCH = 4096
HIST = 200

DPAD = 128
VB = 16384
NBLK = (VOCAB + VB - 1) // VB
VPAD = NBLK * VB

NC, NS = 2, 16
NW = NC * NS

HALF = HIST // 2
N_SLICE = 4
NBAT = BATCH // N_SLICE
BATCH_PER_W = NBAT // NW
SUPER = BATCH_PER_W // 2


def _transpose_pad(tableT):
    def body(in_ref, out_ref):
        out_ref[:, pl.ds(0, DIM)] = in_ref[...].T

    return pl.pallas_call(
        body,
        grid=(NBLK,),
        in_specs=[pl.BlockSpec((DIM, VB), lambda i: (0, i))],
        out_specs=pl.BlockSpec((VB, DPAD), lambda i: (i, 0)),
        out_shape=jax.ShapeDtypeStruct((VPAD, DPAD), jnp.float32),
    )(tableT)


BB, HB = 1024, 8
GB, GH = BATCH // BB, HIST // HB


def _relayout(o):
    def body(in_ref, out_ref):
        for hh in range(HB):
            out_ref[:, hh, :] = in_ref[:, hh, :].T

    oT = pl.pallas_call(
        body,
        grid=(GB, GH),
        in_specs=[pl.BlockSpec((BB, HB, DIM), lambda i, j: (i, j, 0))],
        out_specs=pl.BlockSpec((DIM, HB, BB), lambda i, j: (0, j, i)),
        out_shape=jax.ShapeDtypeStruct((DIM, HIST, BATCH), jnp.float32),
    )(o)
    return jnp.transpose(oT, (2, 1, 0))


def _relayout_slice(acc, o_s, s):
    def body(acc_ref, in_ref, out_ref):
        for hh in range(HB):
            out_ref[:, hh, :] = in_ref[:, hh, :].T

    col0 = s * (NBAT // BB)
    return pl.pallas_call(
        body,
        grid=(NBAT // BB, GH),
        in_specs=[pl.BlockSpec(memory_space=pl.ANY),
                  pl.BlockSpec((BB, HB, DIM), lambda i, j: (i, j, 0))],
        out_specs=pl.BlockSpec((DIM, HB, BB), lambda i, j: (0, j, col0 + i)),
        out_shape=jax.ShapeDtypeStruct((DIM, HIST, BATCH), jnp.float32),
        input_output_aliases={0: 0},
    )(acc, o_s)


def _relayout_first(o_s):
    def body(in_ref, out_ref):
        for hh in range(HB):
            out_ref[:, hh, :] = in_ref[:, hh, :].T

    return pl.pallas_call(
        body,
        grid=(NBAT // BB, GH),
        in_specs=[pl.BlockSpec((BB, HB, DIM), lambda i, j: (i, j, 0))],
        out_specs=pl.BlockSpec((DIM, HB, BB), lambda i, j: (0, j, i)),
        out_shape=jax.ShapeDtypeStruct((DIM, HIST, BATCH), jnp.float32),
    )(o_s)


def _make_gather():
    mesh = plsc.VectorSubcoreMesh(core_axis_name="c", subcore_axis_name="s")

    @pl.kernel(
        out_type=jax.ShapeDtypeStruct((NBAT, HIST, DIM), jnp.float32),
        mesh=mesh,
        scratch_types=[
            pltpu.VMEM((2 * BATCH_PER_W, DPAD), jnp.int32),
            pltpu.VMEM((4, HALF, DPAD), jnp.float32),
            pltpu.VMEM((HIST, DIM), jnp.float32),
            pltpu.SemaphoreType.DMA((4,)),
            pltpu.SemaphoreType.DMA,
        ],
    )
    def emb_kernel(x_hbm, table_hbm, out_hbm, idx_v, gbuf, sbuf, gsem, wsem):
        wid = lax.axis_index("s") * NC + lax.axis_index("c")
        bat0 = wid * BATCH_PER_W
        pltpu.sync_copy(x_hbm.at[pl.ds(2 * bat0, 2 * BATCH_PER_W)], idx_v)
        tail_mask = lax.iota(jnp.int32, 16) >= 12

        def gather(k, half, s):
            pltpu.async_copy(
                table_hbm.at[idx_v.at[2 * k + half, pl.ds(0, HALF)]],
                gbuf.at[s], gsem.at[s])

        def gather_wait(s):
            pltpu.make_async_copy(
                table_hbm.at[idx_v.at[0, pl.ds(0, HALF)]], gbuf.at[s], gsem.at[s]
            ).wait()

        def compact(s, half):
            def rows4(i4, carry):
                for r in range(4):
                    i = i4 * 4 + r
                    o = half * HALF + i
                    for k in range(6):
                        sbuf[o, pl.ds(16 * k, 16)] = gbuf[s, i, pl.ds(16 * k, 16)]
                    tail = gbuf[s, i, pl.ds(84, 16)]
                    cur = sbuf[o, pl.ds(84, 16)]
                    sbuf[o, pl.ds(84, 16)] = jnp.where(tail_mask, tail, cur)
                return carry

            lax.fori_loop(0, HALF // 4, rows4, 0)

        def write(k):
            pltpu.async_copy(sbuf, out_hbm.at[bat0 + k], wsem)

        def write_wait():
            pltpu.make_async_copy(sbuf, out_hbm.at[0], wsem).wait()

        def batch_body(k, b, first, last):
            s0, s1 = 2 * b, 2 * b + 1
            gather_wait(s0)
            if not first:
                write_wait()
            compact(s0, 0)
            if not last:
                gather(k + 2, 0, s0)
            gather_wait(s1)
            compact(s1, 1)
            if not last:
                gather(k + 2, 1, s1)
            write(k)

        for b in range(2):
            for half in range(2):
                gather(b, half, 2 * b + half)
        batch_body(0, 0, True, False)
        batch_body(1, 1, False, False)

        def round_(g, carry):
            for b in range(2):
                batch_body(g * 2 + b, b, False, False)
            return carry

        lax.fori_loop(1, SUPER - 1, round_, 0)

        batch_body(BATCH_PER_W - 2, 0, False, True)
        batch_body(BATCH_PER_W - 1, 1, False, True)
        write_wait()

    return emb_kernel


_gather = _make_gather()


@jax.jit
def kernel(x, table):
    tablep = _transpose_pad(jnp.transpose(table))
    x2 = jnp.pad(x.reshape(2 * BATCH, HALF).astype(jnp.int32),
                 ((0, 0), (0, DPAD - HALF)))
    if N_SLICE == 1:
        return _relayout(_gather(x2, tablep))
    acc = None
    for s in range(N_SLICE):
        o_s = _gather(lax.slice_in_dim(x2, 2 * s * NBAT, 2 * (s + 1) * NBAT),
                      tablep)
        acc = _relayout_first(o_s) if s == 0 else _relayout_slice(acc, o_s, s)
    return jnp.transpose(acc, (2, 1, 0))
